# Initial kernel scaffold; baseline (speedup 1.0000x reference)
import jax
import jax.numpy as jnp
from jax.experimental import pallas as pl


def kernel(x, edge_index, batch, emb_W, emb_b, ln_s, ln_b, pin_W, pin_b, sgu_s, sgu_b, gcn_W, gcn_b, pout_W, pout_b, out_W, out_b):
    raise NotImplementedError("write your pallas kernel here")


# speedup vs baseline: 14.6160x; 14.6160x over previous
import dataclasses
import functools
import math

import jax
import jax.numpy as jnp
from jax import lax
from jax.experimental import pallas as pl
from jax.experimental.pallas import tpu as pltpu
from jax.experimental.pallas import tpu_sc as plsc

_N = 10000
_NPAD = 10240
_E = 320000
_NCORES = 2
_NSUB = 16
_NW = _NCORES * _NSUB
_CH = 128
_EPT = ((_E + _NW * _CH - 1) // (_NW * _CH)) * _CH
_EPAD = _EPT * _NW
_CHH = 2528
_R = 1000
_PREC = lax.Precision.HIGHEST

_sc_mesh = plsc.VectorSubcoreMesh(core_axis_name="c", subcore_axis_name="s")

_sc_params = pltpu.CompilerParams()
if "needs_layout_passes" in pltpu.CompilerParams.__dataclass_fields__:
    _sc_params = dataclasses.replace(_sc_params, needs_layout_passes=False)


@functools.partial(
    pl.kernel,
    out_type=jax.ShapeDtypeStruct((_NW, _NPAD), jnp.float32),
    mesh=_sc_mesh,
    compiler_params=_sc_params,
    scratch_types=[
        pltpu.VMEM((_NPAD,), jnp.float32),
        pltpu.VMEM((_CHH,), jnp.int32),
    ],
)
def _sc_degree_hist(col_hbm, out_hbm, hist_v, cbuf):
    c = lax.axis_index("c")
    s = lax.axis_index("s")
    wid = c * _NSUB + s

    @pl.loop(0, _NPAD // 16)
    def _(i):
        hist_v[pl.ds(i * 16, 16)] = jnp.zeros((16,), jnp.float32)

    ones = jnp.ones((16,), jnp.float32)
    base = wid * _EPT

    @pl.loop(0, _EPT // _CHH)
    def _(j):
        pltpu.sync_copy(col_hbm.at[pl.ds(base + j * _CHH, _CHH)], cbuf)

        @pl.loop(0, _CHH // 16)
        def _(k):
            idx = cbuf[pl.ds(k * 16, 16)]
            plsc.addupdate_scatter(hist_v, [idx], ones)

    pltpu.sync_copy(hist_v, out_hbm.at[wid])


@functools.partial(
    pl.kernel,
    out_type=jax.ShapeDtypeStruct((_NCORES, _NPAD, 128), jnp.float32),
    mesh=_sc_mesh,
    scratch_types=[
        pltpu.VMEM((_CH, 128), jnp.float32),
        pltpu.VMEM((_CH,), jnp.int32),
        pltpu.VMEM((_CH,), jnp.int32),
        pltpu.VMEM_SHARED((_NPAD, 128), jnp.float32),
        pltpu.SemaphoreType.DMA,
    ],
)
def _sc_edge_pass(y_hbm, row_hbm, col_hbm, zeros_hbm, out_hbm,
                  gbuf, rbuf, cbuf, acc, sem):
    c = lax.axis_index("c")
    s = lax.axis_index("s")
    wid = c * _NSUB + s

    pltpu.sync_copy(zeros_hbm, gbuf)
    zrows = _NPAD // _NSUB

    @pl.loop(0, zrows // _CH)
    def _(i):
        pltpu.sync_copy(gbuf, acc.at[pl.ds(s * zrows + i * _CH, _CH)])

    plsc.subcore_barrier()

    base = wid * _EPT

    @pl.loop(0, _EPT // _CH)
    def _(j):
        pltpu.sync_copy(row_hbm.at[pl.ds(base + j * _CH, _CH)], rbuf)
        pltpu.sync_copy(col_hbm.at[pl.ds(base + j * _CH, _CH)], cbuf)
        pltpu.async_copy(y_hbm.at[rbuf], gbuf, sem).wait()
        pltpu.sync_copy(gbuf, acc.at[cbuf], add=True)

    plsc.subcore_barrier()

    orows = _NPAD // _NSUB
    pltpu.sync_copy(acc.at[pl.ds(s * orows, orows)],
                    out_hbm.at[c, pl.ds(s * orows, orows)])


def _ln(x, scale, bias):
    mu = jnp.mean(x, axis=-1, keepdims=True)
    var = jnp.mean((x - mu) ** 2, axis=-1, keepdims=True)
    return (x - mu) * lax.rsqrt(var + 1e-5) * scale + bias


def _full(shape):
    return pl.BlockSpec(shape, lambda i: (0,) * len(shape))


def _rows(shape):
    return pl.BlockSpec(shape, lambda i: (i,) + (0,) * (len(shape) - 1))


def _dinv_body(hist_ref, out_ref):
    total = jnp.sum(hist_ref[...], axis=0) + 2.0
    out_ref[...] = lax.rsqrt(total)[:, None]


def _embed_body(x_ref, w_ref, b_ref, out_ref):
    out_ref[...] = (
        jnp.dot(x_ref[...], w_ref[...], precision=_PREC) + b_ref[...]
    )


def _pre_body(h_ref, dinv_ref, lns_ref, lnb_ref, pinw_ref, pinb_ref,
              sgus_ref, sgub_ref, gcnw_ref, t_ref, y_ref):
    hn = _ln(h_ref[...], lns_ref[...], lnb_ref[...])
    t = jnp.dot(hn, pinw_ref[...], precision=_PREC) + pinb_ref[...]
    t = 0.5 * t * (1.0 + lax.erf(t * (1.0 / math.sqrt(2.0))))
    g = _ln(t, sgus_ref[...], sgub_ref[...])
    y = dinv_ref[...] * jnp.dot(g, gcnw_ref[...], precision=_PREC)
    t_ref[...] = t
    y_ref[...] = y


def _post_body(p_ref, y_ref, t_ref, h_ref, dinv_ref, gcnb_ref,
               poutw_ref, poutb_ref, out_ref):
    acc = p_ref[0] + p_ref[1] + 2.0 * y_ref[...]
    g2 = jnp.tanh(dinv_ref[...] * acc + gcnb_ref[...])
    t2 = g2 * t_ref[...]
    out_ref[...] = (
        h_ref[...]
        + jnp.dot(t2, poutw_ref[...], precision=_PREC)
        + poutb_ref[...]
    )


def _final_body(h_ref, w_ref, b_ref, out_ref):
    out_ref[...] = (
        jnp.dot(h_ref[...], w_ref[...], precision=_PREC) + b_ref[...]
    )


_G = _N // _R


def _tc(body, out_shapes, in_specs, out_specs):
    return pl.pallas_call(
        body,
        grid=(_G,),
        in_specs=in_specs,
        out_specs=out_specs,
        out_shape=out_shapes,
    )


def kernel(x, edge_index, batch, emb_W, emb_b, ln_s, ln_b, pin_W, pin_b,
           sgu_s, sgu_b, gcn_W, gcn_b, pout_W, pout_b, out_W, out_b):
    f32 = jnp.float32
    row = edge_index[0]
    col = edge_index[1]

    npad = _EPAD - _E
    pidx = jnp.arange(npad, dtype=jnp.int32)
    row_p = jnp.concatenate([row, (pidx * 37) % _N])
    col_p = jnp.concatenate([col, _N + pidx % (_NPAD - _N)])
    zeros_blk = jnp.zeros((_CH, 128), f32)

    hist = _sc_degree_hist(col_p)
    dinv = pl.pallas_call(
        _dinv_body,
        out_shape=jax.ShapeDtypeStruct((_NPAD, 1), f32),
    )(hist)
    dinv = dinv[:_N]

    h = _tc(
        _embed_body,
        jax.ShapeDtypeStruct((_N, 128), f32),
        [_rows((_R, 128)), _full((128, 128)), _full((1, 128))],
        _rows((_R, 128)),
    )(x, emb_W.T, emb_b[None, :])

    for i in range(2):
        t, y = _tc(
            _pre_body,
            (jax.ShapeDtypeStruct((_N, 128), f32),
             jax.ShapeDtypeStruct((_N, 128), f32)),
            [_rows((_R, 128)), _rows((_R, 1)),
             _full((1, 128)), _full((1, 128)),
             _full((128, 128)), _full((1, 128)),
             _full((1, 128)), _full((1, 128)),
             _full((128, 128))],
            (_rows((_R, 128)), _rows((_R, 128))),
        )(h, dinv, ln_s[i][None, :], ln_b[i][None, :], pin_W[i].T,
          pin_b[i][None, :], sgu_s[i][None, :], sgu_b[i][None, :],
          gcn_W[i].T)

        partials = _sc_edge_pass(y, row_p, col_p, zeros_blk)

        h = _tc(
            _post_body,
            jax.ShapeDtypeStruct((_N, 128), f32),
            [pl.BlockSpec((2, _R, 128), lambda i: (0, i, 0)),
             _rows((_R, 128)), _rows((_R, 128)), _rows((_R, 128)),
             _rows((_R, 1)), _full((1, 128)),
             _full((128, 128)), _full((1, 128))],
            _rows((_R, 128)),
        )(partials, y, t, h, dinv, gcn_b[i][None, :], pout_W[i].T,
          pout_b[i][None, :])

    out = _tc(
        _final_body,
        jax.ShapeDtypeStruct((_N, 64), f32),
        [_rows((_R, 128)), _full((128, 64)), _full((1, 64))],
        _rows((_R, 64)),
    )(h, out_W.T, out_b[None, :])
    return out


# speedup vs baseline: 24.5226x; 1.6778x over previous
import dataclasses
import functools
import math

import jax
import jax.numpy as jnp
from jax import lax
from jax.experimental import pallas as pl
from jax.experimental.pallas import tpu as pltpu
from jax.experimental.pallas import tpu_sc as plsc

_N = 10000
_NPAD = 10240
_E = 320000
_NCORES = 2
_NSUB = 16
_NW = _NCORES * _NSUB
_CH = 128
_NCH = 80
_EPT = _NCH * _CH
_EPAD = _EPT * _NW
_R = 1000
_PREC = lax.Precision.HIGHEST

_sc_mesh = plsc.VectorSubcoreMesh(core_axis_name="c", subcore_axis_name="s")

_sc_params = pltpu.CompilerParams()
if "needs_layout_passes" in pltpu.CompilerParams.__dataclass_fields__:
    _sc_params = dataclasses.replace(_sc_params, needs_layout_passes=False)


@functools.partial(
    pl.kernel,
    out_type=jax.ShapeDtypeStruct((_NW, _NPAD), jnp.float32),
    mesh=_sc_mesh,
    compiler_params=_sc_params,
    scratch_types=[
        pltpu.VMEM((_NPAD,), jnp.float32),
        pltpu.VMEM((_NCH, _CH), jnp.int32),
    ],
)
def _sc_degree_hist(col_hbm, out_hbm, hist_v, cslab):
    c = lax.axis_index("c")
    s = lax.axis_index("s")
    wid = c * _NSUB + s

    @pl.loop(0, _NPAD // 16)
    def _(i):
        hist_v[pl.ds(i * 16, 16)] = jnp.zeros((16,), jnp.float32)

    ones = jnp.ones((16,), jnp.float32)
    pltpu.sync_copy(col_hbm.at[wid], cslab)

    @pl.loop(0, _NCH)
    def _(j):
        @pl.loop(0, _CH // 16)
        def _(k):
            idx = cslab[j, pl.ds(k * 16, 16)]
            plsc.addupdate_scatter(hist_v, [idx], ones)

    pltpu.sync_copy(hist_v, out_hbm.at[wid])


@functools.partial(
    pl.kernel,
    out_type=jax.ShapeDtypeStruct((_NCORES, _NPAD, 128), jnp.float32),
    mesh=_sc_mesh,
    scratch_types=[
        pltpu.VMEM((_CH, 128), jnp.float32),
        pltpu.VMEM((_CH, 128), jnp.float32),
        pltpu.VMEM((_NCH, _CH), jnp.int32),
        [pltpu.VMEM((_CH,), jnp.int32)] * 4,
        pltpu.SemaphoreType.DMA,
        pltpu.SemaphoreType.DMA,
        [pltpu.SemaphoreType.DMA] * 4,
        pltpu.VMEM_SHARED((_NPAD, 128), jnp.float32),
    ],
)
def _sc_edge_pass(y_hbm, row_hbm, col_hbm, zeros_hbm, out_hbm,
                  gbuf0, gbuf1, cslab, rbufs, sem0, sem1, rsems, acc):
    c = lax.axis_index("c")
    s = lax.axis_index("s")
    wid = c * _NSUB + s

    pltpu.sync_copy(zeros_hbm, gbuf0)
    zrows = _NPAD // _NSUB

    @pl.loop(0, zrows // _CH)
    def _(i):
        pltpu.sync_copy(gbuf0, acc.at[pl.ds(s * zrows + i * _CH, _CH)])

    plsc.subcore_barrier()

    pltpu.sync_copy(col_hbm.at[wid], cslab)

    gbufs = (gbuf0, gbuf1)
    gsems = (sem0, sem1)
    for k in range(4):
        pltpu.async_copy(row_hbm.at[wid, k], rbufs[k], rsems[k])
    for k in range(2):
        pltpu.make_async_copy(row_hbm.at[wid, k], rbufs[k], rsems[k]).wait()
        pltpu.async_copy(y_hbm.at[rbufs[k]], gbufs[k], gsems[k])

    @pl.loop(0, _NCH, step=4)
    def _(j):
        for i in range(4):
            jj = j + i
            b = i % 2
            pltpu.make_async_copy(y_hbm.at[rbufs[i]], gbufs[b], gsems[b]).wait()
            pltpu.sync_copy(gbufs[b], acc.at[cslab.at[jj]], add=True)

            @pl.when(jj + 4 < _NCH)
            def _():
                pltpu.async_copy(row_hbm.at[wid, jj + 4], rbufs[i], rsems[i])

            @pl.when(jj + 2 < _NCH)
            def _():
                r2 = (i + 2) % 4
                pltpu.make_async_copy(
                    row_hbm.at[wid, jj + 2], rbufs[r2], rsems[r2]).wait()
                pltpu.async_copy(y_hbm.at[rbufs[r2]], gbufs[b], gsems[b])

    plsc.subcore_barrier()

    orows = _NPAD // _NSUB
    pltpu.sync_copy(acc.at[pl.ds(s * orows, orows)],
                    out_hbm.at[c, pl.ds(s * orows, orows)])


def _ln(x, scale, bias):
    mu = jnp.mean(x, axis=-1, keepdims=True)
    var = jnp.mean((x - mu) ** 2, axis=-1, keepdims=True)
    return (x - mu) * lax.rsqrt(var + 1e-5) * scale + bias


def _full(shape):
    return pl.BlockSpec(shape, lambda i: (0,) * len(shape))


def _rows(shape):
    return pl.BlockSpec(shape, lambda i: (i,) + (0,) * (len(shape) - 1))


def _dinv_body(hist_ref, out_ref):
    total = jnp.sum(hist_ref[...], axis=0) + 2.0
    out_ref[...] = lax.rsqrt(total)[:, None]


def _embed_body(x_ref, w_ref, b_ref, out_ref):
    out_ref[...] = (
        jnp.dot(x_ref[...], w_ref[...], precision=_PREC) + b_ref[...]
    )


def _pre_body(h_ref, dinv_ref, lns_ref, lnb_ref, pinw_ref, pinb_ref,
              sgus_ref, sgub_ref, gcnw_ref, t_ref, y_ref):
    hn = _ln(h_ref[...], lns_ref[...], lnb_ref[...])
    t = jnp.dot(hn, pinw_ref[...], precision=_PREC) + pinb_ref[...]
    t = 0.5 * t * (1.0 + lax.erf(t * (1.0 / math.sqrt(2.0))))
    g = _ln(t, sgus_ref[...], sgub_ref[...])
    y = dinv_ref[...] * jnp.dot(g, gcnw_ref[...], precision=_PREC)
    t_ref[...] = t
    y_ref[...] = y


def _post_body(p_ref, y_ref, t_ref, h_ref, dinv_ref, gcnb_ref,
               poutw_ref, poutb_ref, out_ref):
    acc = p_ref[0] + p_ref[1] + 2.0 * y_ref[...]
    g2 = jnp.tanh(dinv_ref[...] * acc + gcnb_ref[...])
    t2 = g2 * t_ref[...]
    out_ref[...] = (
        h_ref[...]
        + jnp.dot(t2, poutw_ref[...], precision=_PREC)
        + poutb_ref[...]
    )


def _final_body(h_ref, w_ref, b_ref, out_ref):
    out_ref[...] = (
        jnp.dot(h_ref[...], w_ref[...], precision=_PREC) + b_ref[...]
    )


_G = _N // _R


def _tc(body, out_shapes, in_specs, out_specs):
    return pl.pallas_call(
        body,
        grid=(_G,),
        in_specs=in_specs,
        out_specs=out_specs,
        out_shape=out_shapes,
    )


def kernel(x, edge_index, batch, emb_W, emb_b, ln_s, ln_b, pin_W, pin_b,
           sgu_s, sgu_b, gcn_W, gcn_b, pout_W, pout_b, out_W, out_b):
    f32 = jnp.float32
    row = edge_index[0]
    col = edge_index[1]

    npad = _EPAD - _E
    pidx = jnp.arange(npad, dtype=jnp.int32)
    row_p = jnp.concatenate([row, (pidx * 37) % _N]).reshape(_NW, _NCH, _CH)
    col_p = jnp.concatenate([col, _N + pidx % (_NPAD - _N)])
    col_p = col_p.reshape(_NW, _NCH, _CH)
    zeros_blk = jnp.zeros((_CH, 128), f32)

    hist = _sc_degree_hist(col_p)
    dinv = pl.pallas_call(
        _dinv_body,
        out_shape=jax.ShapeDtypeStruct((_NPAD, 1), f32),
    )(hist)
    dinv = dinv[:_N]

    h = _tc(
        _embed_body,
        jax.ShapeDtypeStruct((_N, 128), f32),
        [_rows((_R, 128)), _full((128, 128)), _full((1, 128))],
        _rows((_R, 128)),
    )(x, emb_W.T, emb_b[None, :])

    for i in range(2):
        t, y = _tc(
            _pre_body,
            (jax.ShapeDtypeStruct((_N, 128), f32),
             jax.ShapeDtypeStruct((_N, 128), f32)),
            [_rows((_R, 128)), _rows((_R, 1)),
             _full((1, 128)), _full((1, 128)),
             _full((128, 128)), _full((1, 128)),
             _full((1, 128)), _full((1, 128)),
             _full((128, 128))],
            (_rows((_R, 128)), _rows((_R, 128))),
        )(h, dinv, ln_s[i][None, :], ln_b[i][None, :], pin_W[i].T,
          pin_b[i][None, :], sgu_s[i][None, :], sgu_b[i][None, :],
          gcn_W[i].T)

        partials = _sc_edge_pass(y, row_p, col_p, zeros_blk)

        h = _tc(
            _post_body,
            jax.ShapeDtypeStruct((_N, 128), f32),
            [pl.BlockSpec((2, _R, 128), lambda i: (0, i, 0)),
             _rows((_R, 128)), _rows((_R, 128)), _rows((_R, 128)),
             _rows((_R, 1)), _full((1, 128)),
             _full((128, 128)), _full((1, 128))],
            _rows((_R, 128)),
        )(partials, y, t, h, dinv, gcn_b[i][None, :], pout_W[i].T,
          pout_b[i][None, :])

    out = _tc(
        _final_body,
        jax.ShapeDtypeStruct((_N, 64), f32),
        [_rows((_R, 128)), _full((128, 64)), _full((1, 64))],
        _rows((_R, 64)),
    )(h, out_W.T, out_b[None, :])
    return out


# speedup vs baseline: 29.0776x; 1.1857x over previous
import dataclasses
import functools
import math

import numpy as np

import jax
import jax.numpy as jnp
from jax import lax
from jax.experimental import pallas as pl
from jax.experimental.pallas import tpu as pltpu
from jax.experimental.pallas import tpu_sc as plsc

_N = 10000
_NPAD = 10240
_E = 320000
_NCORES = 2
_NSUB = 16
_NW = _NCORES * _NSUB
_CH = 128
_NCH = 80
_EPT = _NCH * _CH
_EPAD = _EPT * _NW
_R = 1000
_PREC = lax.Precision.DEFAULT
_DN = (((1,), (1,)), ((), ()))


def _matmul(a, w):
    return lax.dot_general(a, w, _DN, precision=_PREC)

_sc_mesh = plsc.VectorSubcoreMesh(core_axis_name="c", subcore_axis_name="s")

_sc_params = pltpu.CompilerParams()
if "needs_layout_passes" in pltpu.CompilerParams.__dataclass_fields__:
    _sc_params = dataclasses.replace(_sc_params, needs_layout_passes=False)


@functools.partial(
    pl.kernel,
    out_type=jax.ShapeDtypeStruct((_NW, _NPAD), jnp.float32),
    mesh=_sc_mesh,
    compiler_params=_sc_params,
    scratch_types=[
        pltpu.VMEM((_NPAD,), jnp.float32),
        pltpu.VMEM((_NCH, _CH), jnp.int32),
    ],
)
def _sc_degree_hist(col_hbm, out_hbm, hist_v, cslab):
    c = lax.axis_index("c")
    s = lax.axis_index("s")
    wid = c * _NSUB + s

    @pl.loop(0, _NPAD // 16)
    def _(i):
        hist_v[pl.ds(i * 16, 16)] = jnp.zeros((16,), jnp.float32)

    ones = jnp.ones((16,), jnp.float32)
    pltpu.sync_copy(col_hbm.at[wid], cslab)

    @pl.loop(0, _NCH)
    def _(j):
        @pl.loop(0, _CH // 16)
        def _(k):
            idx = cslab[j, pl.ds(k * 16, 16)]
            plsc.addupdate_scatter(hist_v, [idx], ones)

    pltpu.sync_copy(hist_v, out_hbm.at[wid])


@functools.partial(
    pl.kernel,
    out_type=jax.ShapeDtypeStruct((_NCORES, _NPAD, 128), jnp.float32),
    mesh=_sc_mesh,
    scratch_types=[
        pltpu.VMEM((_CH, 128), jnp.float32),
        pltpu.VMEM((_CH, 128), jnp.float32),
        pltpu.VMEM((_NCH, _CH), jnp.int32),
        [pltpu.VMEM((_CH,), jnp.int32)] * 4,
        pltpu.SemaphoreType.DMA,
        pltpu.SemaphoreType.DMA,
        [pltpu.SemaphoreType.DMA] * 4,
        pltpu.VMEM_SHARED((_NPAD, 128), jnp.float32),
    ],
)
def _sc_edge_pass(y_hbm, row_hbm, col_hbm, zeros_hbm, out_hbm,
                  gbuf0, gbuf1, cslab, rbufs, sem0, sem1, rsems, acc):
    c = lax.axis_index("c")
    s = lax.axis_index("s")
    wid = c * _NSUB + s

    pltpu.sync_copy(zeros_hbm, gbuf0)
    zrows = _NPAD // _NSUB

    @pl.loop(0, zrows // _CH)
    def _(i):
        pltpu.sync_copy(gbuf0, acc.at[pl.ds(s * zrows + i * _CH, _CH)])

    plsc.subcore_barrier()

    pltpu.sync_copy(col_hbm.at[wid], cslab)

    gbufs = (gbuf0, gbuf1)
    gsems = (sem0, sem1)
    for k in range(4):
        pltpu.async_copy(row_hbm.at[wid, k], rbufs[k], rsems[k])
    for k in range(2):
        pltpu.make_async_copy(row_hbm.at[wid, k], rbufs[k], rsems[k]).wait()
        pltpu.async_copy(y_hbm.at[rbufs[k]], gbufs[k], gsems[k])

    @pl.loop(0, _NCH, step=4)
    def _(j):
        for i in range(4):
            jj = j + i
            b = i % 2
            pltpu.make_async_copy(y_hbm.at[rbufs[i]], gbufs[b], gsems[b]).wait()
            pltpu.sync_copy(gbufs[b], acc.at[cslab.at[jj]], add=True)

            @pl.when(jj + 4 < _NCH)
            def _():
                pltpu.async_copy(row_hbm.at[wid, jj + 4], rbufs[i], rsems[i])

            @pl.when(jj + 2 < _NCH)
            def _():
                r2 = (i + 2) % 4
                pltpu.make_async_copy(
                    row_hbm.at[wid, jj + 2], rbufs[r2], rsems[r2]).wait()
                pltpu.async_copy(y_hbm.at[rbufs[r2]], gbufs[b], gsems[b])

    plsc.subcore_barrier()

    orows = _NPAD // _NSUB
    pltpu.sync_copy(acc.at[pl.ds(s * orows, orows)],
                    out_hbm.at[c, pl.ds(s * orows, orows)])


def _ln(x, scale, bias):
    mu = jnp.mean(x, axis=-1, keepdims=True)
    var = jnp.mean((x - mu) ** 2, axis=-1, keepdims=True)
    return (x - mu) * lax.rsqrt(var + 1e-5) * scale + bias


def _full(shape):
    return pl.BlockSpec(shape, lambda i: (0,) * len(shape))


def _rows(shape):
    return pl.BlockSpec(shape, lambda i: (i,) + (0,) * (len(shape) - 1))


def _dinv_body(hist_ref, out_ref):
    total = jnp.sum(hist_ref[...], axis=0) + 2.0
    out_ref[...] = lax.rsqrt(total)[:, None]


def _embed_body(x_ref, w_ref, b_ref, out_ref):
    out_ref[...] = _matmul(x_ref[...], w_ref[...]) + b_ref[...]


def _pre_body(h_ref, dinv_ref, lns_ref, lnb_ref, pinw_ref, pinb_ref,
              sgus_ref, sgub_ref, gcnw_ref, t_ref, y_ref):
    hn = _ln(h_ref[...], lns_ref[...], lnb_ref[...])
    t = _matmul(hn, pinw_ref[...]) + pinb_ref[...]
    t = 0.5 * t * (1.0 + lax.erf(t * (1.0 / math.sqrt(2.0))))
    g = _ln(t, sgus_ref[...], sgub_ref[...])
    y = dinv_ref[...] * _matmul(g, gcnw_ref[...])
    t_ref[...] = t
    y_ref[...] = y


def _post_body(p_ref, y_ref, t_ref, h_ref, dinv_ref, gcnb_ref,
               poutw_ref, poutb_ref, out_ref):
    acc = p_ref[0] + p_ref[1] + 2.0 * y_ref[...]
    g2 = jnp.tanh(dinv_ref[...] * acc + gcnb_ref[...])
    t2 = g2 * t_ref[...]
    out_ref[...] = h_ref[...] + _matmul(t2, poutw_ref[...]) + poutb_ref[...]


def _final_body(h_ref, w_ref, b_ref, out_ref):
    out_ref[...] = _matmul(h_ref[...], w_ref[...]) + b_ref[...]


_G = _N // _R


def _tc(body, out_shapes, in_specs, out_specs):
    return pl.pallas_call(
        body,
        grid=(_G,),
        in_specs=in_specs,
        out_specs=out_specs,
        out_shape=out_shapes,
    )


def kernel(x, edge_index, batch, emb_W, emb_b, ln_s, ln_b, pin_W, pin_b,
           sgu_s, sgu_b, gcn_W, gcn_b, pout_W, pout_b, out_W, out_b):
    f32 = jnp.float32
    row = edge_index[0]
    col = edge_index[1]

    npad = _EPAD - _E
    pidx = np.arange(npad, dtype=np.int32)
    row_p = jnp.concatenate(
        [row, jnp.asarray((pidx * 37) % _N, jnp.int32)]).reshape(_NW, _NCH, _CH)
    col_p = jnp.concatenate(
        [col, jnp.asarray(_N + pidx % (_NPAD - _N), jnp.int32)]
    ).reshape(_NW, _NCH, _CH)
    zeros_blk = jnp.zeros((_CH, 128), f32)

    hist = _sc_degree_hist(col_p)
    dinv = pl.pallas_call(
        _dinv_body,
        out_shape=jax.ShapeDtypeStruct((_NPAD, 1), f32),
    )(hist)

    h = _tc(
        _embed_body,
        jax.ShapeDtypeStruct((_N, 128), f32),
        [_rows((_R, 128)), _full((128, 128)), _full((1, 128))],
        _rows((_R, 128)),
    )(x, emb_W, emb_b[None, :])

    for i in range(2):
        t, y = _tc(
            _pre_body,
            (jax.ShapeDtypeStruct((_N, 128), f32),
             jax.ShapeDtypeStruct((_N, 128), f32)),
            [_rows((_R, 128)), _rows((_R, 1)),
             _full((1, 128)), _full((1, 128)),
             _full((128, 128)), _full((1, 128)),
             _full((1, 128)), _full((1, 128)),
             _full((128, 128))],
            (_rows((_R, 128)), _rows((_R, 128))),
        )(h, dinv, ln_s[i][None, :], ln_b[i][None, :], pin_W[i],
          pin_b[i][None, :], sgu_s[i][None, :], sgu_b[i][None, :],
          gcn_W[i])

        partials = _sc_edge_pass(y, row_p, col_p, zeros_blk)

        h = _tc(
            _post_body,
            jax.ShapeDtypeStruct((_N, 128), f32),
            [pl.BlockSpec((2, _R, 128), lambda i: (0, i, 0)),
             _rows((_R, 128)), _rows((_R, 128)), _rows((_R, 128)),
             _rows((_R, 1)), _full((1, 128)),
             _full((128, 128)), _full((1, 128))],
            _rows((_R, 128)),
        )(partials, y, t, h, dinv, gcn_b[i][None, :], pout_W[i],
          pout_b[i][None, :])

    out = _tc(
        _final_body,
        jax.ShapeDtypeStruct((_N, 64), f32),
        [_rows((_R, 128)), _full((64, 128)), _full((1, 64))],
        _rows((_R, 64)),
    )(h, out_W, out_b[None, :])
    return out
